# Initial kernel scaffold; baseline (speedup 1.0000x reference)
import jax
import jax.numpy as jnp
from jax.experimental import pallas as pl


def kernel(support_pair, support_path, path_len, support_path_entity, support_relation_set, ent_emb, rel_emb, W_d, b_d, W_e, b_e, W_u, b_u):
    raise NotImplementedError("write your pallas kernel here")


# speedup vs baseline: 2.3614x; 2.3614x over previous
import functools

import jax
import jax.numpy as jnp
from jax import lax
from jax.experimental import pallas as pl
from jax.experimental.pallas import tpu as pltpu
from jax.experimental.pallas import tpu_sc as plsc

_D = 128
_NW = 32
_LANES = 16

_META_W = 144
_OFF_PLEN = 48
_OFF_REL = 64
_OFF_ENT = 96
_N_REL_G = 32
_N_ENT_G = 48


def _rsqrt(x):
    i = lax.bitcast_convert_type(x, jnp.int32)
    i = jnp.int32(0x5F3759DF) - lax.shift_right_logical(i, 1)
    y = lax.bitcast_convert_type(i, jnp.float32)
    for _ in range(3):
        y = y * (1.5 - 0.5 * x * y * y)
    return y


def _prep_body(rel_ref, wd_ref, we_ref, wu_ref, bu_ref, out_ref):
    x = rel_ref[...]
    dn = (((1,), (1,)), ((), ()))
    hi = jax.lax.Precision.HIGHEST
    out_ref[:, 0:128] = lax.dot_general(x, wd_ref[...], dn, precision=hi,
                                        preferred_element_type=jnp.float32)
    out_ref[:, 128:256] = lax.dot_general(x, we_ref[...], dn, precision=hi,
                                          preferred_element_type=jnp.float32)
    uu = lax.dot_general(x, wu_ref[...], dn, precision=hi,
                         preferred_element_type=jnp.float32)
    out_ref[:, 256:384] = jnp.broadcast_to(uu[:, 0:1] + bu_ref[0, 0],
                                           (128, 128))


def _build_table(rel_emb, W_d, W_e, W_u, b_u):
    n = rel_emb.shape[0]
    npad = ((n + 127) // 128) * 128
    relp = jnp.pad(rel_emb, ((0, npad - n), (0, 0)))
    wu8 = jnp.pad(W_u, ((0, 7), (0, 0)))
    bub = jnp.broadcast_to(b_u.reshape(1, 1), (8, 128))
    return pl.pallas_call(
        _prep_body,
        grid=(npad // 128,),
        in_specs=[
            pl.BlockSpec((128, _D), lambda i: (i, 0)),
            pl.BlockSpec((_D, _D), lambda i: (0, 0)),
            pl.BlockSpec((_D, _D), lambda i: (0, 0)),
            pl.BlockSpec((8, _D), lambda i: (0, 0)),
            pl.BlockSpec((8, _D), lambda i: (0, 0)),
        ],
        out_specs=pl.BlockSpec((128, 3 * _D), lambda i: (i, 0)),
        out_shape=jax.ShapeDtypeStruct((npad, 3 * _D), jnp.float32),
    )(relp, W_d, W_e, wu8, bub)


def _sc_body(meta_hbm, ent_hbm, tab_hbm, bias_hbm, out_hbm, th_hbm,
             meta_v, ent_v, rel_v, out_v, th_v, att_v, bias_v, dsl_v, ssl_v,
             sem):
    n_pairs = meta_hbm.shape[0]
    per_w = n_pairs // _NW
    wid = lax.axis_index("s") * 2 + lax.axis_index("c")
    base = wid * per_w

    pltpu.sync_copy(bias_hbm, bias_v)
    iota = lax.iota(jnp.int32, _LANES)

    def pair_body(i, carry):
        b = base + i
        pltpu.sync_copy(meta_hbm.at[b], meta_v)
        pltpu.async_copy(ent_hbm.at[meta_v.at[pl.ds(_OFF_ENT, _N_ENT_G)]],
                         ent_v, sem).wait()
        pltpu.async_copy(tab_hbm.at[meta_v.at[pl.ds(_OFF_REL, _N_REL_G)]],
                         rel_v, sem).wait()

        nh = jnp.zeros((_LANES,), jnp.float32)
        nt = jnp.zeros((_LANES,), jnp.float32)
        ht = jnp.zeros((_LANES,), jnp.float32)
        for c in range(8):
            sl = pl.ds(16 * c, 16)
            vh = ent_v[0, sl]
            vt = ent_v[1, sl]
            nh = nh + vh * vh
            nt = nt + vt * vt
            ht = ht + vh * vt
            th_v[sl] = vt - vh
        nh_s = jnp.sum(nh)
        nt_s = jnp.sum(nt)
        ht_s = jnp.sum(ht)

        def path_body(p, carry2):
            a1 = jnp.zeros((_LANES,), jnp.float32)
            a2 = jnp.zeros((_LANES,), jnp.float32)
            a3 = jnp.zeros((_LANES,), jnp.float32)
            a4 = jnp.zeros((_LANES,), jnp.float32)
            a5 = jnp.zeros((_LANES,), jnp.float32)
            a6 = jnp.zeros((_LANES,), jnp.float32)
            for c in range(8):
                sl = pl.ds(16 * c, 16)
                vh = ent_v[0, sl]
                vt = ent_v[1, sl]
                v1 = ent_v[2 * p + 2, sl]
                v2 = ent_v[2 * p + 3, sl]
                a1 = a1 + v1 * v1
                a2 = a2 + vh * v1
                a3 = a3 + vt * v1
                a4 = a4 + v2 * v2
                a5 = a5 + vh * v2
                a6 = a6 + vt * v2
            pm = iota == p
            return (jnp.where(pm, jnp.sum(a1), carry2[0]),
                    jnp.where(pm, jnp.sum(a2), carry2[1]),
                    jnp.where(pm, jnp.sum(a3), carry2[2]),
                    jnp.where(pm, jnp.sum(a4), carry2[3]),
                    jnp.where(pm, jnp.sum(a5), carry2[4]),
                    jnp.where(pm, jnp.sum(a6), carry2[5]))

        zv = jnp.zeros((_LANES,), jnp.float32)
        n1v, hd1, td1, n2v, hd2, td2 = lax.fori_loop(
            0, 16, path_body, (zv, zv, zv, zv, zv, zv))

        nhb = jnp.full((_LANES,), nh_s)
        ntb = jnp.full((_LANES,), nt_s)
        eps = jnp.float32(1e-16)
        s1v = 0.5 * (hd1 * _rsqrt(jnp.maximum(nhb * n1v, eps))
                     + td1 * _rsqrt(jnp.maximum(ntb * n1v, eps)))
        s2v = 0.5 * (hd2 * _rsqrt(jnp.maximum(nhb * n2v, eps))
                     + td2 * _rsqrt(jnp.maximum(ntb * n2v, eps)))
        shv = ht_s * _rsqrt(jnp.maximum(nhb * ntb, eps))
        lv = meta_v[pl.ds(_OFF_PLEN, 16)] + 1
        one = jnp.float32(1.0)
        sc3 = jnp.float32(0.001)
        zero = jnp.float32(0.0)
        dks = (jnp.where(lv == 1, one - shv, one - 0.5 * (shv + s1v)),
               jnp.where(lv == 2, one - 0.5 * (shv + s1v),
                         one - 0.5 * (s1v + s2v)),
               one - 0.5 * (s2v + shv))
        scs = (jnp.full((_LANES,), sc3),
               jnp.where(lv >= 2, sc3, zero),
               jnp.where(lv >= 3, sc3, zero))

        col = jnp.full((_LANES,), 256, jnp.int32)
        u0 = plsc.load_gather(rel_v, [iota, col])
        rows1 = jnp.minimum(iota + 16, 19)
        u1 = plsc.load_gather(rel_v, [rows1, col])
        m1 = iota < 4
        u1m = jnp.where(m1, u1, jnp.float32(-1e30))
        m = jnp.max(jnp.maximum(u0, u1m))
        e0 = jnp.exp(u0 - m)
        e1 = jnp.where(m1, jnp.exp(u1 - m), zero)
        s = jnp.sum(e0 + e1)
        att_v[pl.ds(0, 16)] = e0 / s
        att_v[pl.ds(16, 16)] = e1 / s

        plsc.store_scatter(dsl_v, [iota * 3 + 0], dks[0])
        plsc.store_scatter(dsl_v, [iota * 3 + 1], dks[1])
        plsc.store_scatter(dsl_v, [iota * 3 + 2], dks[2])
        plsc.store_scatter(ssl_v, [iota * 3 + 0], scs[0])
        plsc.store_scatter(ssl_v, [iota * 3 + 1], scs[1])
        plsc.store_scatter(ssl_v, [iota * 3 + 2], scs[2])
        for g in range(3):
            jv = meta_v[pl.ds(16 * g, 16)]
            av = plsc.load_gather(att_v, [jv])
            dcv = dsl_v[pl.ds(16 * g, 16)]
            scv = ssl_v[pl.ds(16 * g, 16)]
            rowv = iota + 16 * g
            for c in range(8):
                bv = bias_v[pl.ds(16 * c, 16)]
                for l in range(16):
                    dd = 16 * c + l
                    colv = jnp.full((_LANES,), dd, jnp.int32)
                    ydv = plsc.load_gather(rel_v, [jv, colv])
                    yev = plsc.load_gather(rel_v, [jv, colv + 128])
                    val = dcv * ydv + av * yev + bv[l]
                    plsc.store_scatter(
                        out_v, [rowv, colv],
                        jnp.maximum(val, 0.01 * val) * scv)

        pltpu.sync_copy(out_v, out_hbm.at[b])
        pltpu.sync_copy(th_v, th_hbm.at[b])
        return carry

    lax.fori_loop(0, per_w, pair_body, 0)


def kernel(support_pair, support_path, path_len, support_path_entity,
           support_relation_set, ent_emb, rel_emb, W_d, b_d, W_e, b_e,
           W_u, b_u):
    B, P, _ = support_path.shape
    i32 = jnp.int32
    sp = support_pair.astype(i32)
    spath = support_path.astype(i32).reshape(B, 3 * P)
    plen = path_len.astype(i32)
    spe = support_path_entity.astype(i32).reshape(B, 2 * P)
    rels = support_relation_set.astype(i32)

    meta = jnp.concatenate(
        [spath, plen, rels,
         jnp.zeros((B, _N_REL_G - rels.shape[1]), i32), sp, spe,
         jnp.zeros((B, _N_ENT_G - 2 - 2 * P), i32)],
        axis=1)

    table = _build_table(rel_emb.astype(jnp.float32), W_d, W_e, W_u, b_u)
    bias = (b_d + b_e).astype(jnp.float32)

    mesh = plsc.VectorSubcoreMesh(core_axis_name="c", subcore_axis_name="s",
                                  num_cores=2, num_subcores=16)
    out, th = pl.kernel(
        _sc_body,
        out_type=(jax.ShapeDtypeStruct((B, 3 * P, _D), jnp.float32),
                  jax.ShapeDtypeStruct((B, _D), jnp.float32)),
        mesh=mesh,
        compiler_params=pltpu.CompilerParams(needs_layout_passes=False),
        scratch_types=[
            pltpu.VMEM((_META_W,), i32),
            pltpu.VMEM((_N_ENT_G, _D), jnp.float32),
            pltpu.VMEM((_N_REL_G, 3 * _D), jnp.float32),
            pltpu.VMEM((3 * P, _D), jnp.float32),
            pltpu.VMEM((_D,), jnp.float32),
            pltpu.VMEM((32,), jnp.float32),
            pltpu.VMEM((_D,), jnp.float32),
            pltpu.VMEM((3 * P,), jnp.float32),
            pltpu.VMEM((3 * P,), jnp.float32),
            pltpu.SemaphoreType.DMA,
        ],
    )(meta, ent_emb.astype(jnp.float32), table, bias)
    return out.reshape(B, P, 3, _D), th
